# Initial kernel scaffold; baseline (speedup 1.0000x reference)
import jax
import jax.numpy as jnp
from jax.experimental import pallas as pl


def kernel(z, pos, batch, emb, mlp_w1, mlp_b1, mlp_w2, mlp_b2, lin1_w, lin2_w, lin2_b, lin_w, lin_b, out1_w, out1_b, out2_w, out2_b):
    raise NotImplementedError("write your pallas kernel here")


# speedup vs baseline: 10.3480x; 10.3480x over previous
import functools

import jax
import jax.numpy as jnp
from jax import lax
from jax.experimental import pallas as pl
from jax.experimental.pallas import tpu as pltpu

N_ATOMS = 8192
N_GRAPHS = 512
HIDDEN = 128
FILTERS = 128
N_INTER = 3
N_GAUSS = 50
NGP = 64
CUTOFF = 10.0
CH = 64
N_CHUNKS = N_ATOMS // CH
ROWS = 1024
LOG2 = 0.6931471805599453


def _ssp(x):
    return jnp.maximum(x, 0.0) + jnp.log1p(jnp.exp(-jnp.abs(x))) - LOG2


def _embed_body(z_ref, emb_ref, out_ref):
    onehot = (z_ref[...] == lax.broadcasted_iota(jnp.int32, (1, 128), 1))
    out_ref[...] = jnp.dot(onehot.astype(jnp.float32), emb_ref[...],
                           preferred_element_type=jnp.float32)


def _xf_body(h_ref, w_ref, out_ref):
    out_ref[...] = jnp.dot(h_ref[...], w_ref[...],
                           preferred_element_type=jnp.float32)


def _update_body(h_ref, agg_ref, l2w_ref, l2b_ref, lw_ref, lb_ref, out_ref):
    v = _ssp(jnp.dot(agg_ref[...], l2w_ref[...],
                     preferred_element_type=jnp.float32) + l2b_ref[...])
    v = jnp.dot(v, lw_ref[...], preferred_element_type=jnp.float32) + lb_ref[...]
    out_ref[...] = h_ref[...] + v


def _pair_body(meta_ref, p_i_ref, sq_i_ref, b_i_ref, xf3_ref, pos3_ref,
               sq3_ref, b3_ref, offs_ref, coeff_ref, w1_ref, b1_ref, w2_ref,
               b2_ref, agg_ref):
    c = pl.program_id(0)
    t0 = meta_ref[c, 0]
    nt = meta_ref[c, 1]
    i0 = c * CH
    p_i = p_i_ref[...]
    sq_i = sq_i_ref[...]
    b_i = b_i_ref[...]
    row_ids = i0 + lax.broadcasted_iota(jnp.int32, (CH, 1), 0)
    coeff = coeff_ref[0, 0]
    offs3 = offs_ref[...][None]
    kmask3 = (lax.broadcasted_iota(jnp.int32, (1, 1, NGP), 2) < N_GAUSS)
    cut2 = CUTOFF * CUTOFF

    def body(s, acc):
        t = t0 + s
        j0 = t * CH
        p_j = pos3_ref[t]
        sq_j = sq3_ref[t]
        b_j = b3_ref[t]
        xf_j = xf3_ref[t]
        col_ids = j0 + lax.broadcasted_iota(jnp.int32, (1, CH), 1)
        cross = lax.dot_general(p_i, p_j, (((1,), (1,)), ((), ())),
                                preferred_element_type=jnp.float32)
        d2 = sq_i + sq_j - 2.0 * cross
        mask = (d2 < cut2) & (b_i == b_j) & (row_ids != col_ids)
        diff = p_i[:, None, :] - p_j[None, :, :]
        ew = jnp.sqrt(jnp.sum(diff * diff, axis=-1) + 1e-12)
        g = jnp.exp(coeff * (ew[:, :, None] - offs3) ** 2)
        g = jnp.where(kmask3, g, 0.0)
        attr = g.reshape(CH * CH, NGP)
        w = _ssp(jnp.dot(attr, w1_ref[...],
                         preferred_element_type=jnp.float32) + b1_ref[...])
        w = jnp.dot(w, w2_ref[...],
                    preferred_element_type=jnp.float32) + b2_ref[...]
        cc = 0.5 * (jnp.cos(ew * (jnp.pi / CUTOFF)) + 1.0)
        scale = jnp.where(mask, cc, 0.0)
        w3 = w.reshape(CH, CH, FILTERS) * scale[:, :, None]
        acc = acc + jnp.sum(w3 * xf_j[None, :, :], axis=1)
        return acc

    agg_ref[...] = lax.fori_loop(
        0, nt, body, jnp.zeros((CH, FILTERS), jnp.float32))


def _readout_body(h_ref, bc_ref, w1_ref, b1_ref, w2r_ref, b2_ref, out_ref):
    c = pl.program_id(0)
    i0 = c * CH
    hh = _ssp(jnp.dot(h_ref[...], w1_ref[...],
                      preferred_element_type=jnp.float32) + b1_ref[...])
    e = jnp.sum(hh * w2r_ref[...], axis=1, keepdims=True) + b2_ref[0, 0]
    b_i = bc_ref[pl.ds(i0, CH), :]
    onehot = (b_i == lax.broadcasted_iota(jnp.int32, (CH, N_GRAPHS), 1))
    contrib = lax.dot_general(e, onehot.astype(jnp.float32),
                              (((0,), (0,)), ((), ())),
                              preferred_element_type=jnp.float32)

    @pl.when(c == 0)
    def _():
        out_ref[...] = jnp.zeros_like(out_ref)

    out_ref[...] += contrib


def _full(shape):
    return pl.BlockSpec(shape, lambda c: (0,) * len(shape))


def kernel(z, pos, batch, emb, mlp_w1, mlp_b1, mlp_w2, mlp_b2, lin1_w,
           lin2_w, lin2_b, lin_w, lin_b, out1_w, out1_b, out2_w, out2_b):
    batch = batch.astype(jnp.int32)
    z = z.astype(jnp.int32)

    pos_pad = jnp.zeros((N_ATOMS, 8), jnp.float32).at[:, :3].set(pos)
    sq = jnp.sum(pos * pos, axis=1)
    sq_col = sq[:, None]
    sq_row = sq[None, :]
    b_col = batch[:, None]
    b_row = batch[None, :]

    gids = jnp.arange(N_GRAPHS, dtype=batch.dtype)
    seg_start = jnp.searchsorted(batch, gids, side="left").astype(jnp.int32)
    seg_end = jnp.searchsorted(batch, gids, side="right").astype(jnp.int32)
    bmat = batch.reshape(N_CHUNKS, CH)
    lo = seg_start[bmat[:, 0]]
    hi = seg_end[bmat[:, CH - 1]]
    t0 = lo // CH
    nt = (hi + CH - 1) // CH - t0
    meta = jnp.stack([t0, nt], axis=1).astype(jnp.int32)

    offsets = jnp.linspace(0.0, CUTOFF, N_GAUSS)
    coeff = (-0.5 / (offsets[1] - offsets[0]) ** 2).reshape(1, 1)
    offs_pad = jnp.zeros((1, NGP), jnp.float32).at[0, :N_GAUSS].set(offsets)
    w1_pad = jnp.zeros((N_INTER, NGP, FILTERS), jnp.float32)
    w1_pad = w1_pad.at[:, :N_GAUSS, :].set(mlp_w1)
    emb_pad = jnp.zeros((128, HIDDEN), jnp.float32).at[:100, :].set(emb)

    h = pl.pallas_call(
        _embed_body,
        grid=(N_CHUNKS,),
        in_specs=[pl.BlockSpec((CH, 1), lambda c: (c, 0)),
                  _full((128, HIDDEN))],
        out_specs=pl.BlockSpec((CH, HIDDEN), lambda c: (c, 0)),
        out_shape=jax.ShapeDtypeStruct((N_ATOMS, HIDDEN), jnp.float32),
    )(z[:, None], emb_pad)

    n_rb = N_ATOMS // ROWS
    for t in range(N_INTER):
        xf = pl.pallas_call(
            _xf_body,
            grid=(n_rb,),
            in_specs=[pl.BlockSpec((ROWS, HIDDEN), lambda c: (c, 0)),
                      _full((HIDDEN, FILTERS))],
            out_specs=pl.BlockSpec((ROWS, FILTERS), lambda c: (c, 0)),
            out_shape=jax.ShapeDtypeStruct((N_ATOMS, FILTERS), jnp.float32),
        )(h, lin1_w[t])

        agg = pl.pallas_call(
            _pair_body,
            grid=(N_CHUNKS,),
            in_specs=[pl.BlockSpec(memory_space=pltpu.SMEM),
                      pl.BlockSpec((CH, 8), lambda c: (c, 0)),
                      pl.BlockSpec((CH, 1), lambda c: (c, 0)),
                      pl.BlockSpec((CH, 1), lambda c: (c, 0)),
                      _full((N_CHUNKS, CH, FILTERS)),
                      _full((N_CHUNKS, CH, 8)),
                      _full((N_CHUNKS, 1, CH)),
                      _full((N_CHUNKS, 1, CH)),
                      _full((1, NGP)),
                      pl.BlockSpec(memory_space=pltpu.SMEM),
                      _full((NGP, FILTERS)),
                      _full((1, FILTERS)),
                      _full((FILTERS, FILTERS)),
                      _full((1, FILTERS))],
            out_specs=pl.BlockSpec((CH, FILTERS), lambda c: (c, 0)),
            out_shape=jax.ShapeDtypeStruct((N_ATOMS, FILTERS), jnp.float32),
        )(meta, pos_pad, sq_col, b_col, xf.reshape(N_CHUNKS, CH, FILTERS),
          pos_pad.reshape(N_CHUNKS, CH, 8), sq.reshape(N_CHUNKS, 1, CH),
          batch.reshape(N_CHUNKS, 1, CH), offs_pad,
          coeff, w1_pad[t], mlp_b1[t][None, :], mlp_w2[t],
          mlp_b2[t][None, :])

        h = pl.pallas_call(
            _update_body,
            grid=(n_rb,),
            in_specs=[pl.BlockSpec((ROWS, HIDDEN), lambda c: (c, 0)),
                      pl.BlockSpec((ROWS, FILTERS), lambda c: (c, 0)),
                      _full((FILTERS, HIDDEN)),
                      _full((1, HIDDEN)),
                      _full((HIDDEN, HIDDEN)),
                      _full((1, HIDDEN))],
            out_specs=pl.BlockSpec((ROWS, HIDDEN), lambda c: (c, 0)),
            out_shape=jax.ShapeDtypeStruct((N_ATOMS, HIDDEN), jnp.float32),
        )(h, agg, lin2_w[t], lin2_b[t][None, :], lin_w[t], lin_b[t][None, :])

    energy = pl.pallas_call(
        _readout_body,
        grid=(N_CHUNKS,),
        in_specs=[pl.BlockSpec((CH, HIDDEN), lambda c: (c, 0)),
                  _full((N_ATOMS, 1)),
                  _full((HIDDEN, HIDDEN // 2)),
                  _full((1, HIDDEN // 2)),
                  _full((1, HIDDEN // 2)),
                  _full((1, 1))],
        out_specs=_full((1, N_GRAPHS)),
        out_shape=jax.ShapeDtypeStruct((1, N_GRAPHS), jnp.float32),
    )(h, b_col, out1_w, out1_b[None, :], out2_w.reshape(1, HIDDEN // 2),
      out2_b.reshape(1, 1))

    return energy.reshape(N_GRAPHS, 1)
